# Initial kernel scaffold; baseline (speedup 1.0000x reference)
import jax
import jax.numpy as jnp
from jax.experimental import pallas as pl


def kernel(nodes, edges, senders, receivers, enc_msg_W0, enc_msg_b0, enc_msg_W1, enc_msg_b1, enc_ln_m_s, enc_ln_m_b, enc_node_W0, enc_node_b0, enc_node_W1, enc_node_b1, enc_ln_n_s, enc_ln_n_b, prc_msg_W0, prc_msg_b0, prc_msg_W1, prc_msg_b1, prc_ln_m_s, prc_ln_m_b, prc_node_W0, prc_node_b0, prc_node_W1, prc_node_b1, prc_ln_n_s, prc_ln_n_b):
    raise NotImplementedError("write your pallas kernel here")


# speedup vs baseline: 1.7775x; 1.7775x over previous
import functools

import jax
import jax.numpy as jnp
from jax import lax
from jax.experimental import pallas as pl
from jax.experimental.pallas import tpu as pltpu
from jax.experimental.pallas import tpu_sc as plsc

NC, NS = 2, 16
NW = NC * NS
G = 80


def _layernorm(x, s, b):
    mu = jnp.mean(x, axis=-1, keepdims=True)
    var = jnp.mean((x - mu) ** 2, axis=-1, keepdims=True)
    return (x - mu) * lax.rsqrt(var + 1e-6) * s + b


def _make_gather(N, D, E):
    EW = E // NW
    NCH = EW // G
    mesh = plsc.VectorSubcoreMesh(core_axis_name="c", subcore_axis_name="s")

    @functools.partial(
        pl.kernel,
        out_type=(jax.ShapeDtypeStruct((E, D), jnp.float32),
                  jax.ShapeDtypeStruct((E, D), jnp.float32)),
        mesh=mesh,
        scratch_types=[
            pltpu.VMEM((NCH, G), jnp.int32),
            pltpu.VMEM((NCH, G), jnp.int32),
            pltpu.VMEM((G, D), jnp.float32),
            pltpu.VMEM((G, D), jnp.float32),
            pltpu.SemaphoreType.DMA,
            pltpu.SemaphoreType.DMA,
        ],
    )
    def gather_k(nodes_hbm, sidx_hbm, ridx_hbm, s_out, r_out,
                 siv, riv, bufs, bufr, sems, semr):
        wid = lax.axis_index("s") * NC + lax.axis_index("c")
        base = wid * EW
        pltpu.sync_copy(sidx_hbm.at[wid], siv)
        pltpu.sync_copy(ridx_hbm.at[wid], riv)

        def body(j, carry):
            pltpu.async_copy(nodes_hbm.at[siv.at[j]], bufs, sems).wait()
            pltpu.sync_copy(bufs, s_out.at[pl.ds(base + j * G, G)])
            pltpu.async_copy(nodes_hbm.at[riv.at[j]], bufr, semr).wait()
            pltpu.sync_copy(bufr, r_out.at[pl.ds(base + j * G, G)])
            return carry

        lax.fori_loop(0, NCH, body, 0)

    return gather_k


def _make_scatter(D, E, NPAD):
    EW = E // NW
    NCH = EW // G
    RPT = NPAD // NS
    mesh = plsc.VectorSubcoreMesh(core_axis_name="c", subcore_axis_name="s")

    @functools.partial(
        pl.kernel,
        out_type=(jax.ShapeDtypeStruct((NPAD, D), jnp.float32),
                  jax.ShapeDtypeStruct((NPAD, D), jnp.float32)),
        mesh=mesh,
        scratch_types=[
            pltpu.VMEM((NCH, G), jnp.int32),
            pltpu.VMEM((G, D), jnp.float32),
            pltpu.VMEM_SHARED((NPAD, D), jnp.float32),
        ],
    )
    def scatter_k(msg_hbm, ridx_hbm, zeros_hbm, agg0, agg1, riv, mbuf, shared):
        cid = lax.axis_index("c")
        sid = lax.axis_index("s")
        wid = sid * NC + cid
        pltpu.sync_copy(zeros_hbm.at[pl.ds(sid * RPT, RPT)],
                        shared.at[pl.ds(sid * RPT, RPT)])
        plsc.subcore_barrier()
        pltpu.sync_copy(ridx_hbm.at[wid], riv)

        def body(j, carry):
            pltpu.sync_copy(msg_hbm.at[pl.ds(wid * EW + j * G, G)], mbuf)
            pltpu.sync_copy(mbuf, shared.at[riv.at[j]], add=True)
            return carry

        lax.fori_loop(0, NCH, body, 0)
        plsc.subcore_barrier()

        @pl.when(cid == 0)
        def _():
            pltpu.sync_copy(shared.at[pl.ds(sid * RPT, RPT)],
                            agg0.at[pl.ds(sid * RPT, RPT)])

        @pl.when(cid == 1)
        def _():
            pltpu.sync_copy(shared.at[pl.ds(sid * RPT, RPT)],
                            agg1.at[pl.ds(sid * RPT, RPT)])

    return scatter_k


def _make_edge_mlp(E, D, DE, H, BE):
    prec = lax.Precision.HIGHEST

    def body(s_ref, r_ref, e_ref, w0a, w0b, w0c, b0, w1, b1, lns, lnb, out_ref):
        x = jnp.dot(s_ref[...], w0a[...], preferred_element_type=jnp.float32,
                    precision=prec)
        x = x + jnp.dot(r_ref[...], w0b[...],
                        preferred_element_type=jnp.float32, precision=prec)
        x = x + jnp.dot(e_ref[...], w0c[...],
                        preferred_element_type=jnp.float32, precision=prec)
        x = jnp.maximum(x + b0[...], 0.0)
        m = jnp.dot(x, w1[...], preferred_element_type=jnp.float32,
                    precision=prec) + b1[...]
        out_ref[...] = _layernorm(m, lns[...], lnb[...])

    rep = lambda i: (0, 0)
    return pl.pallas_call(
        body,
        grid=(E // BE,),
        in_specs=[
            pl.BlockSpec((BE, D), lambda i: (i, 0)),
            pl.BlockSpec((BE, D), lambda i: (i, 0)),
            pl.BlockSpec((BE, DE), lambda i: (i, 0)),
            pl.BlockSpec((D, H), rep),
            pl.BlockSpec((D, H), rep),
            pl.BlockSpec((DE, H), rep),
            pl.BlockSpec((1, H), rep),
            pl.BlockSpec((H, H), rep),
            pl.BlockSpec((1, H), rep),
            pl.BlockSpec((1, H), rep),
            pl.BlockSpec((1, H), rep),
        ],
        out_specs=pl.BlockSpec((BE, H), lambda i: (i, 0)),
        out_shape=jax.ShapeDtypeStruct((E, H), jnp.float32),
    )


def _make_node_mlp(N, D, H, BN):
    prec = lax.Precision.HIGHEST

    def body(n_ref, a0_ref, a1_ref, w0a, w0b, b0, w1, b1, lns, lnb, out_ref):
        agg = a0_ref[...] + a1_ref[...]
        x = jnp.dot(n_ref[...], w0a[...], preferred_element_type=jnp.float32,
                    precision=prec)
        x = x + jnp.dot(agg, w0b[...], preferred_element_type=jnp.float32,
                        precision=prec)
        x = jnp.maximum(x + b0[...], 0.0)
        u = jnp.dot(x, w1[...], preferred_element_type=jnp.float32,
                    precision=prec) + b1[...]
        out_ref[...] = n_ref[...] + _layernorm(u, lns[...], lnb[...])

    rep = lambda i: (0, 0)
    return pl.pallas_call(
        body,
        grid=(N // BN,),
        in_specs=[
            pl.BlockSpec((BN, D), lambda i: (i, 0)),
            pl.BlockSpec((BN, D), lambda i: (i, 0)),
            pl.BlockSpec((BN, D), lambda i: (i, 0)),
            pl.BlockSpec((D, H), rep),
            pl.BlockSpec((H, H), rep),
            pl.BlockSpec((1, H), rep),
            pl.BlockSpec((H, D), rep),
            pl.BlockSpec((1, D), rep),
            pl.BlockSpec((1, D), rep),
            pl.BlockSpec((1, D), rep),
        ],
        out_specs=pl.BlockSpec((BN, D), lambda i: (i, 0)),
        out_shape=jax.ShapeDtypeStruct((N, D), jnp.float32),
    )


def kernel(nodes, edges, senders, receivers,
           enc_msg_W0, enc_msg_b0, enc_msg_W1, enc_msg_b1, enc_ln_m_s, enc_ln_m_b,
           enc_node_W0, enc_node_b0, enc_node_W1, enc_node_b1, enc_ln_n_s, enc_ln_n_b,
           prc_msg_W0, prc_msg_b0, prc_msg_W1, prc_msg_b1, prc_ln_m_s, prc_ln_m_b,
           prc_node_W0, prc_node_b0, prc_node_W1, prc_node_b1, prc_ln_n_s, prc_ln_n_b):
    N, D = nodes.shape
    E, DE = edges.shape
    H = enc_msg_W0.shape[1]

    EW = E // NW
    NCH = EW // G
    RPT = -(-N // NS)
    RPT = ((RPT + 7) // 8) * 8
    NPAD = RPT * NS

    s3 = senders.reshape(NW, NCH, G)
    r3 = receivers.reshape(NW, NCH, G)
    zeros = jnp.zeros((NPAD, D), jnp.float32)

    gather_k = _make_gather(N, D, E)
    scatter_k = _make_scatter(D, E, NPAD)
    edge_k = _make_edge_mlp(E, D, DE, H, BE=4000)
    node_k = _make_node_mlp(N, D, H, BN=2000)

    def layer(h, msg_W0, msg_b0, msg_W1, msg_b1, ln_m_s, ln_m_b,
              node_W0, node_b0, node_W1, node_b1, ln_n_s, ln_n_b):
        w0a, w0b, w0c = msg_W0[:D], msg_W0[D:2 * D], msg_W0[2 * D:]
        S, R = gather_k(h, s3, r3)
        M = edge_k(S, R, edges, w0a, w0b, w0c, msg_b0.reshape(1, H),
                   msg_W1, msg_b1.reshape(1, H),
                   ln_m_s.reshape(1, H), ln_m_b.reshape(1, H))
        agg0, agg1 = scatter_k(M, r3, zeros)
        return node_k(h, agg0, agg1, node_W0[:D], node_W0[D:],
                      node_b0.reshape(1, H), node_W1, node_b1.reshape(1, D),
                      ln_n_s.reshape(1, D), ln_n_b.reshape(1, D))

    h = layer(nodes, enc_msg_W0, enc_msg_b0, enc_msg_W1, enc_msg_b1,
              enc_ln_m_s, enc_ln_m_b, enc_node_W0, enc_node_b0,
              enc_node_W1, enc_node_b1, enc_ln_n_s, enc_ln_n_b)
    h = layer(h, prc_msg_W0, prc_msg_b0, prc_msg_W1, prc_msg_b1,
              prc_ln_m_s, prc_ln_m_b, prc_node_W0, prc_node_b0,
              prc_node_W1, prc_node_b1, prc_ln_n_s, prc_ln_n_b)
    return h


# speedup vs baseline: 3.8236x; 2.1512x over previous
import functools

import jax
import jax.numpy as jnp
from jax import lax
from jax.experimental import pallas as pl
from jax.experimental.pallas import tpu as pltpu
from jax.experimental.pallas import tpu_sc as plsc

NC, NS = 2, 16
NW = NC * NS
G = 80
KG = 2


def _layernorm(x, s, b):
    mu = jnp.mean(x, axis=-1, keepdims=True)
    var = jnp.mean((x - mu) ** 2, axis=-1, keepdims=True)
    return (x - mu) * lax.rsqrt(var + 1e-6) * s + b


def _make_gather(N, D, E):
    EW = E // NS
    NCH = EW // G
    NG = NCH // KG
    GR = KG * G

    mesh = plsc.VectorSubcoreMesh(core_axis_name="c", subcore_axis_name="s")

    @functools.partial(
        pl.kernel,
        out_type=jax.ShapeDtypeStruct((2, E, D), jnp.float32),
        mesh=mesh,
        scratch_types=[
            pltpu.VMEM((NCH, G), jnp.int32),
            pltpu.VMEM((2, GR, D), jnp.float32),
            pltpu.SemaphoreType.DMA,
            pltpu.SemaphoreType.DMA,
        ],
    )
    def gather_k(nodes_hbm, idx_hbm, sr_out, idxv, big, sem_g, sem_o):
        cid = lax.axis_index("c")
        sid = lax.axis_index("s")
        base = sid * EW
        pltpu.sync_copy(idx_hbm.at[cid, sid], idxv)

        def body(g, carry):
            cur = lax.rem(g, 2)
            descs = []
            for k in range(KG):
                descs.append(pltpu.async_copy(
                    nodes_hbm.at[idxv.at[g * KG + k]],
                    big.at[cur, pl.ds(k * G, G)],
                    sem_g,
                ))
            for d in descs:
                d.wait()

            @pl.when(g >= 1)
            def _():
                pltpu.make_async_copy(
                    big.at[cur], sr_out.at[cid, pl.ds(base, GR)], sem_o,
                ).wait()

            pltpu.async_copy(
                big.at[cur],
                sr_out.at[cid, pl.ds(base + g * GR, GR)],
                sem_o,
            )
            return carry

        lax.fori_loop(0, NG, body, 0)
        pltpu.make_async_copy(
            big.at[0], sr_out.at[cid, pl.ds(base, GR)], sem_o,
        ).wait()

    return gather_k


def _make_scatter(D, E, NPAD):
    EW = E // NW
    NCH = EW // G
    RPT = NPAD // NS
    mesh = plsc.VectorSubcoreMesh(core_axis_name="c", subcore_axis_name="s")

    @functools.partial(
        pl.kernel,
        out_type=(jax.ShapeDtypeStruct((NPAD, D), jnp.float32),
                  jax.ShapeDtypeStruct((NPAD, D), jnp.float32)),
        mesh=mesh,
        scratch_types=[
            pltpu.VMEM((2, G), jnp.int32),
            pltpu.VMEM((2, G, D), jnp.float32),
            pltpu.VMEM_SHARED((NPAD, D), jnp.float32),
            pltpu.SemaphoreType.DMA,
            pltpu.SemaphoreType.DMA,
            pltpu.SemaphoreType.DMA,
        ],
    )
    def scatter_k(msg_hbm, ridx_hbm, zeros_hbm, agg0, agg1,
                  rivc, big, shared, sem_l, sem_sc0, sem_sc1):
        cid = lax.axis_index("c")
        sid = lax.axis_index("s")
        wid = sid * NC + cid
        base = wid * EW
        pltpu.sync_copy(zeros_hbm.at[pl.ds(sid * RPT, RPT)],
                        shared.at[pl.ds(sid * RPT, RPT)])
        plsc.subcore_barrier()

        def _load(g, buf):
            pltpu.async_copy(msg_hbm.at[pl.ds(base + g * G, G)],
                             big.at[buf], sem_l)
            pltpu.async_copy(ridx_hbm.at[wid, g], rivc.at[buf], sem_l)

        def _wait_load(buf):
            pltpu.make_async_copy(
                msg_hbm.at[pl.ds(base, G)], big.at[buf], sem_l).wait()
            pltpu.make_async_copy(
                ridx_hbm.at[wid, 0], rivc.at[buf], sem_l).wait()

        def _drain(sem):
            pltpu.make_async_copy(
                big.at[0], shared.at[rivc.at[0]], sem).wait()

        _load(0, 0)

        def body(g, carry):
            cur = lax.rem(g, 2)
            nxt = 1 - cur
            _wait_load(cur)

            @pl.when(cur == 0)
            def _():
                pltpu.async_copy(big.at[cur], shared.at[rivc.at[cur]],
                                 sem_sc0, add=True)

            @pl.when(cur == 1)
            def _():
                pltpu.async_copy(big.at[cur], shared.at[rivc.at[cur]],
                                 sem_sc1, add=True)

            @pl.when(jnp.logical_and(g >= 1, nxt == 0))
            def _():
                _drain(sem_sc0)

            @pl.when(jnp.logical_and(g >= 1, nxt == 1))
            def _():
                _drain(sem_sc1)

            @pl.when(g + 1 < NCH)
            def _():
                _load(g + 1, nxt)
            return carry

        lax.fori_loop(0, NCH, body, 0)
        if (NCH - 1) % 2 == 0:
            _drain(sem_sc0)
        else:
            _drain(sem_sc1)
        plsc.subcore_barrier()

        @pl.when(cid == 0)
        def _():
            pltpu.sync_copy(shared.at[pl.ds(sid * RPT, RPT)],
                            agg0.at[pl.ds(sid * RPT, RPT)])

        @pl.when(cid == 1)
        def _():
            pltpu.sync_copy(shared.at[pl.ds(sid * RPT, RPT)],
                            agg1.at[pl.ds(sid * RPT, RPT)])

    return scatter_k


def _make_edge_mlp(E, D, DE, H, BE):
    prec = lax.Precision.DEFAULT

    def body(s_ref, r_ref, e_ref, w0a, w0b, w0c, b0, w1, b1, lns, lnb, out_ref):
        x = jnp.dot(s_ref[0], w0a[...], preferred_element_type=jnp.float32,
                    precision=prec)
        x = x + jnp.dot(r_ref[0], w0b[...],
                        preferred_element_type=jnp.float32, precision=prec)
        x = x + jnp.dot(e_ref[...], w0c[...],
                        preferred_element_type=jnp.float32, precision=prec)
        x = jnp.maximum(x + b0[...], 0.0)
        m = jnp.dot(x, w1[...], preferred_element_type=jnp.float32,
                    precision=prec) + b1[...]
        out_ref[...] = _layernorm(m, lns[...], lnb[...])

    rep = lambda i: (0, 0)
    return pl.pallas_call(
        body,
        grid=(E // BE,),
        in_specs=[
            pl.BlockSpec((1, BE, D), lambda i: (0, i, 0)),
            pl.BlockSpec((1, BE, D), lambda i: (1, i, 0)),
            pl.BlockSpec((BE, DE), lambda i: (i, 0)),
            pl.BlockSpec((D, H), rep),
            pl.BlockSpec((D, H), rep),
            pl.BlockSpec((DE, H), rep),
            pl.BlockSpec((1, H), rep),
            pl.BlockSpec((H, H), rep),
            pl.BlockSpec((1, H), rep),
            pl.BlockSpec((1, H), rep),
            pl.BlockSpec((1, H), rep),
        ],
        out_specs=pl.BlockSpec((BE, H), lambda i: (i, 0)),
        out_shape=jax.ShapeDtypeStruct((E, H), jnp.float32),
    )


def _make_node_mlp(N, D, H, BN):
    prec = lax.Precision.DEFAULT

    def body(n_ref, a0_ref, a1_ref, w0a, w0b, b0, w1, b1, lns, lnb, out_ref):
        agg = a0_ref[...] + a1_ref[...]
        x = jnp.dot(n_ref[...], w0a[...], preferred_element_type=jnp.float32,
                    precision=prec)
        x = x + jnp.dot(agg, w0b[...], preferred_element_type=jnp.float32,
                        precision=prec)
        x = jnp.maximum(x + b0[...], 0.0)
        u = jnp.dot(x, w1[...], preferred_element_type=jnp.float32,
                    precision=prec) + b1[...]
        out_ref[...] = n_ref[...] + _layernorm(u, lns[...], lnb[...])

    rep = lambda i: (0, 0)
    return pl.pallas_call(
        body,
        grid=(N // BN,),
        in_specs=[
            pl.BlockSpec((BN, D), lambda i: (i, 0)),
            pl.BlockSpec((BN, D), lambda i: (i, 0)),
            pl.BlockSpec((BN, D), lambda i: (i, 0)),
            pl.BlockSpec((D, H), rep),
            pl.BlockSpec((H, H), rep),
            pl.BlockSpec((1, H), rep),
            pl.BlockSpec((H, D), rep),
            pl.BlockSpec((1, D), rep),
            pl.BlockSpec((1, D), rep),
            pl.BlockSpec((1, D), rep),
        ],
        out_specs=pl.BlockSpec((BN, D), lambda i: (i, 0)),
        out_shape=jax.ShapeDtypeStruct((N, D), jnp.float32),
    )


def kernel(nodes, edges, senders, receivers,
           enc_msg_W0, enc_msg_b0, enc_msg_W1, enc_msg_b1, enc_ln_m_s, enc_ln_m_b,
           enc_node_W0, enc_node_b0, enc_node_W1, enc_node_b1, enc_ln_n_s, enc_ln_n_b,
           prc_msg_W0, prc_msg_b0, prc_msg_W1, prc_msg_b1, prc_ln_m_s, prc_ln_m_b,
           prc_node_W0, prc_node_b0, prc_node_W1, prc_node_b1, prc_ln_n_s, prc_ln_n_b):
    N, D = nodes.shape
    E, DE = edges.shape
    H = enc_msg_W0.shape[1]

    EW = E // NW
    NCH = EW // G
    RPT = -(-N // NS)
    RPT = ((RPT + 7) // 8) * 8
    NPAD = RPT * NS

    idx4 = jnp.stack([senders, receivers]).reshape(2, NS, (E // NS) // G, G)
    r3 = receivers.reshape(NW, NCH, G)
    zeros = jnp.zeros((NPAD, D), jnp.float32)

    gather_k = _make_gather(N, D, E)
    scatter_k = _make_scatter(D, E, NPAD)
    edge_k = _make_edge_mlp(E, D, DE, H, BE=4000)
    node_k = _make_node_mlp(N, D, H, BN=2000)

    def layer(h, msg_W0, msg_b0, msg_W1, msg_b1, ln_m_s, ln_m_b,
              node_W0, node_b0, node_W1, node_b1, ln_n_s, ln_n_b):
        w0a, w0b, w0c = msg_W0[:D], msg_W0[D:2 * D], msg_W0[2 * D:]
        SR = gather_k(h, idx4)
        M = edge_k(SR, SR, edges, w0a, w0b, w0c, msg_b0.reshape(1, H),
                   msg_W1, msg_b1.reshape(1, H),
                   ln_m_s.reshape(1, H), ln_m_b.reshape(1, H))
        agg0, agg1 = scatter_k(M, r3, zeros)
        return node_k(h, agg0, agg1, node_W0[:D], node_W0[D:],
                      node_b0.reshape(1, H), node_W1, node_b1.reshape(1, D),
                      ln_n_s.reshape(1, D), ln_n_b.reshape(1, D))

    h = layer(nodes, enc_msg_W0, enc_msg_b0, enc_msg_W1, enc_msg_b1,
              enc_ln_m_s, enc_ln_m_b, enc_node_W0, enc_node_b0,
              enc_node_W1, enc_node_b1, enc_ln_n_s, enc_ln_n_b)
    h = layer(h, prc_msg_W0, prc_msg_b0, prc_msg_W1, prc_msg_b1,
              prc_ln_m_s, prc_ln_m_b, prc_node_W0, prc_node_b0,
              prc_node_W1, prc_node_b1, prc_ln_n_s, prc_ln_n_b)
    return h


# speedup vs baseline: 4.2578x; 1.1136x over previous
import functools

import jax
import jax.numpy as jnp
from jax import lax
from jax.experimental import pallas as pl
from jax.experimental.pallas import tpu as pltpu
from jax.experimental.pallas import tpu_sc as plsc

NC, NS = 2, 16
NW = NC * NS
G = 80
KG = 2


def _layernorm(x, s, b):
    mu = jnp.mean(x, axis=-1, keepdims=True)
    var = jnp.mean((x - mu) ** 2, axis=-1, keepdims=True)
    return (x - mu) * lax.rsqrt(var + 1e-6) * s + b


def _dot(a, b):
    return jnp.dot(a, b, preferred_element_type=jnp.float32)


def _make_gather(N, D, E):
    EW = E // NS
    NCH = EW // G
    NG = NCH // KG
    GR = KG * G

    mesh = plsc.VectorSubcoreMesh(core_axis_name="c", subcore_axis_name="s")

    @functools.partial(
        pl.kernel,
        out_type=jax.ShapeDtypeStruct((2, E, D), jnp.float32),
        mesh=mesh,
        scratch_types=[
            pltpu.VMEM((NCH, G), jnp.int32),
            pltpu.VMEM((2, GR, D), jnp.float32),
            pltpu.SemaphoreType.DMA,
            pltpu.SemaphoreType.DMA,
        ],
    )
    def gather_k(p_hbm, idx_hbm, sr_out, idxv, big, sem_g, sem_o):
        cid = lax.axis_index("c")
        sid = lax.axis_index("s")
        base = sid * EW
        pltpu.sync_copy(idx_hbm.at[cid, sid], idxv)
        src = p_hbm.at[cid]

        def body(g, carry):
            cur = lax.rem(g, 2)
            descs = []
            for k in range(KG):
                descs.append(pltpu.async_copy(
                    src.at[idxv.at[g * KG + k]],
                    big.at[cur, pl.ds(k * G, G)],
                    sem_g,
                ))
            for d in descs:
                d.wait()

            @pl.when(g >= 1)
            def _():
                pltpu.make_async_copy(
                    big.at[cur], sr_out.at[cid, pl.ds(base, GR)], sem_o,
                ).wait()

            pltpu.async_copy(
                big.at[cur],
                sr_out.at[cid, pl.ds(base + g * GR, GR)],
                sem_o,
            )
            return carry

        lax.fori_loop(0, NG, body, 0)
        pltpu.make_async_copy(
            big.at[0], sr_out.at[cid, pl.ds(base, GR)], sem_o,
        ).wait()

    return gather_k


def _make_scatter(D, E, NPAD):
    EW = E // NW
    NCH = EW // G
    RPT = NPAD // NS
    mesh = plsc.VectorSubcoreMesh(core_axis_name="c", subcore_axis_name="s")

    @functools.partial(
        pl.kernel,
        out_type=(jax.ShapeDtypeStruct((NPAD, D), jnp.float32),
                  jax.ShapeDtypeStruct((NPAD, D), jnp.float32)),
        mesh=mesh,
        scratch_types=[
            pltpu.VMEM((2, G), jnp.int32),
            pltpu.VMEM((2, G, D), jnp.float32),
            pltpu.VMEM_SHARED((NPAD, D), jnp.float32),
            pltpu.SemaphoreType.DMA,
            pltpu.SemaphoreType.DMA,
            pltpu.SemaphoreType.DMA,
        ],
    )
    def scatter_k(msg_hbm, ridx_hbm, zeros_hbm, agg0, agg1,
                  rivc, big, shared, sem_l, sem_sc0, sem_sc1):
        cid = lax.axis_index("c")
        sid = lax.axis_index("s")
        wid = sid * NC + cid
        base = wid * EW
        pltpu.sync_copy(zeros_hbm.at[pl.ds(sid * RPT, RPT)],
                        shared.at[pl.ds(sid * RPT, RPT)])
        plsc.subcore_barrier()

        def _load(g, buf):
            pltpu.async_copy(msg_hbm.at[pl.ds(base + g * G, G)],
                             big.at[buf], sem_l)
            pltpu.async_copy(ridx_hbm.at[wid, g], rivc.at[buf], sem_l)

        def _wait_load(buf):
            pltpu.make_async_copy(
                msg_hbm.at[pl.ds(base, G)], big.at[buf], sem_l).wait()
            pltpu.make_async_copy(
                ridx_hbm.at[wid, 0], rivc.at[buf], sem_l).wait()

        def _drain(sem):
            pltpu.make_async_copy(
                big.at[0], shared.at[rivc.at[0]], sem).wait()

        _load(0, 0)

        def body(g, carry):
            cur = lax.rem(g, 2)
            nxt = 1 - cur
            _wait_load(cur)

            @pl.when(cur == 0)
            def _():
                pltpu.async_copy(big.at[cur], shared.at[rivc.at[cur]],
                                 sem_sc0, add=True)

            @pl.when(cur == 1)
            def _():
                pltpu.async_copy(big.at[cur], shared.at[rivc.at[cur]],
                                 sem_sc1, add=True)

            @pl.when(jnp.logical_and(g >= 1, nxt == 0))
            def _():
                _drain(sem_sc0)

            @pl.when(jnp.logical_and(g >= 1, nxt == 1))
            def _():
                _drain(sem_sc1)

            @pl.when(g + 1 < NCH)
            def _():
                _load(g + 1, nxt)
            return carry

        lax.fori_loop(0, NCH, body, 0)
        if (NCH - 1) % 2 == 0:
            _drain(sem_sc0)
        else:
            _drain(sem_sc1)
        plsc.subcore_barrier()

        @pl.when(cid == 0)
        def _():
            pltpu.sync_copy(shared.at[pl.ds(sid * RPT, RPT)],
                            agg0.at[pl.ds(sid * RPT, RPT)])

        @pl.when(cid == 1)
        def _():
            pltpu.sync_copy(shared.at[pl.ds(sid * RPT, RPT)],
                            agg1.at[pl.ds(sid * RPT, RPT)])

    return scatter_k


def _make_precompute(N, D, H, BN):
    def body(n_ref, w0a, w0b, b0, out_ref):
        n = n_ref[...]
        out_ref[0] = _dot(n, w0a[...]) + b0[...]
        out_ref[1] = _dot(n, w0b[...])

    rep = lambda i: (0, 0)
    return pl.pallas_call(
        body,
        grid=(N // BN,),
        in_specs=[
            pl.BlockSpec((BN, D), lambda i: (i, 0)),
            pl.BlockSpec((D, H), rep),
            pl.BlockSpec((D, H), rep),
            pl.BlockSpec((1, H), rep),
        ],
        out_specs=pl.BlockSpec((2, BN, H), lambda i: (0, i, 0)),
        out_shape=jax.ShapeDtypeStruct((2, N, H), jnp.float32),
    )


def _make_edge_mlp(E, D, DE, H, BE):
    def body(s_ref, r_ref, et_ref, w0c, w1, b1, lns, lnb, out_ref):
        x = s_ref[0] + r_ref[0]
        x = x + lax.dot_general(et_ref[...], w0c[...],
                                (((0,), (0,)), ((), ())),
                                preferred_element_type=jnp.float32)
        x = jnp.maximum(x, 0.0)
        m = _dot(x, w1[...]) + b1[...]
        out_ref[...] = _layernorm(m, lns[...], lnb[...])

    rep = lambda i: (0, 0)
    return pl.pallas_call(
        body,
        grid=(E // BE,),
        in_specs=[
            pl.BlockSpec((1, BE, H), lambda i: (0, i, 0)),
            pl.BlockSpec((1, BE, H), lambda i: (1, i, 0)),
            pl.BlockSpec((DE, BE), lambda i: (0, i)),
            pl.BlockSpec((DE, H), rep),
            pl.BlockSpec((H, H), rep),
            pl.BlockSpec((1, H), rep),
            pl.BlockSpec((1, H), rep),
            pl.BlockSpec((1, H), rep),
        ],
        out_specs=pl.BlockSpec((BE, H), lambda i: (i, 0)),
        out_shape=jax.ShapeDtypeStruct((E, H), jnp.float32),
    )


def _make_node_mlp(N, D, H, BN, with_p):
    def body(n_ref, a0_ref, a1_ref, w0a, w0b, b0, w1, b1, lns, lnb, *rest):
        agg = a0_ref[...] + a1_ref[...]
        x = _dot(n_ref[...], w0a[...]) + _dot(agg, w0b[...])
        x = jnp.maximum(x + b0[...], 0.0)
        u = _dot(x, w1[...]) + b1[...]
        h = n_ref[...] + _layernorm(u, lns[...], lnb[...])
        if with_p:
            pw0a, pw0b, pb0, out_ref, p_ref = rest
            p_ref[0] = _dot(h, pw0a[...]) + pb0[...]
            p_ref[1] = _dot(h, pw0b[...])
        else:
            (out_ref,) = rest
        out_ref[...] = h

    rep = lambda i: (0, 0)
    in_specs = [
        pl.BlockSpec((BN, D), lambda i: (i, 0)),
        pl.BlockSpec((BN, D), lambda i: (i, 0)),
        pl.BlockSpec((BN, D), lambda i: (i, 0)),
        pl.BlockSpec((D, H), rep),
        pl.BlockSpec((H, H), rep),
        pl.BlockSpec((1, H), rep),
        pl.BlockSpec((H, D), rep),
        pl.BlockSpec((1, D), rep),
        pl.BlockSpec((1, D), rep),
        pl.BlockSpec((1, D), rep),
    ]
    out_specs = pl.BlockSpec((BN, D), lambda i: (i, 0))
    out_shape = jax.ShapeDtypeStruct((N, D), jnp.float32)
    if with_p:
        in_specs += [
            pl.BlockSpec((D, H), rep),
            pl.BlockSpec((D, H), rep),
            pl.BlockSpec((1, H), rep),
        ]
        out_specs = [out_specs, pl.BlockSpec((2, BN, H), lambda i: (0, i, 0))]
        out_shape = [out_shape, jax.ShapeDtypeStruct((2, N, H), jnp.float32)]
    return pl.pallas_call(
        body,
        grid=(N // BN,),
        in_specs=in_specs,
        out_specs=out_specs,
        out_shape=out_shape,
    )


def kernel(nodes, edges, senders, receivers,
           enc_msg_W0, enc_msg_b0, enc_msg_W1, enc_msg_b1, enc_ln_m_s, enc_ln_m_b,
           enc_node_W0, enc_node_b0, enc_node_W1, enc_node_b1, enc_ln_n_s, enc_ln_n_b,
           prc_msg_W0, prc_msg_b0, prc_msg_W1, prc_msg_b1, prc_ln_m_s, prc_ln_m_b,
           prc_node_W0, prc_node_b0, prc_node_W1, prc_node_b1, prc_ln_n_s, prc_ln_n_b):
    N, D = nodes.shape
    E, DE = edges.shape
    H = enc_msg_W0.shape[1]

    EW = E // NW
    NCH = EW // G
    RPT = -(-N // NS)
    RPT = ((RPT + 7) // 8) * 8
    NPAD = RPT * NS

    idx4 = jnp.stack([senders, receivers]).reshape(2, NS, (E // NS) // G, G)
    r3 = receivers.reshape(NW, NCH, G)
    zeros = jnp.zeros((NPAD, D), jnp.float32)
    edgesT = edges.T

    pre_k = _make_precompute(N, D, H, BN=2000)
    gather_k = _make_gather(N, D, E)
    scatter_k = _make_scatter(D, E, NPAD)
    edge_k = _make_edge_mlp(E, D, DE, H, BE=6400)
    node_k = _make_node_mlp(N, D, H, 2000, with_p=False)
    node_kp = _make_node_mlp(N, D, H, 2000, with_p=True)

    P1 = pre_k(nodes, enc_msg_W0[:D], enc_msg_W0[D:2 * D],
               enc_msg_b0.reshape(1, H))
    SR = gather_k(P1, idx4)
    M = edge_k(SR, SR, edgesT, enc_msg_W0[2 * D:], enc_msg_W1,
               enc_msg_b1.reshape(1, H), enc_ln_m_s.reshape(1, H),
               enc_ln_m_b.reshape(1, H))
    agg0, agg1 = scatter_k(M, r3, zeros)
    h, P2 = node_kp(nodes, agg0, agg1, enc_node_W0[:D], enc_node_W0[D:],
                    enc_node_b0.reshape(1, H), enc_node_W1,
                    enc_node_b1.reshape(1, D), enc_ln_n_s.reshape(1, D),
                    enc_ln_n_b.reshape(1, D),
                    prc_msg_W0[:D], prc_msg_W0[D:2 * D],
                    prc_msg_b0.reshape(1, H))

    SR2 = gather_k(P2, idx4)
    M2 = edge_k(SR2, SR2, edgesT, prc_msg_W0[2 * D:], prc_msg_W1,
                prc_msg_b1.reshape(1, H), prc_ln_m_s.reshape(1, H),
                prc_ln_m_b.reshape(1, H))
    agg0b, agg1b = scatter_k(M2, r3, zeros)
    return node_k(h, agg0b, agg1b, prc_node_W0[:D], prc_node_W0[D:],
                  prc_node_b0.reshape(1, H), prc_node_W1,
                  prc_node_b1.reshape(1, D), prc_ln_n_s.reshape(1, D),
                  prc_ln_n_b.reshape(1, D))


# speedup vs baseline: 4.6841x; 1.1001x over previous
import functools

import jax
import jax.numpy as jnp
from jax import lax
from jax.experimental import pallas as pl
from jax.experimental.pallas import tpu as pltpu
from jax.experimental.pallas import tpu_sc as plsc

NC, NS = 2, 16
NW = NC * NS
G = 80
KG = 2


def _layernorm(x, s, b):
    mu = jnp.mean(x, axis=-1, keepdims=True)
    var = jnp.mean((x - mu) ** 2, axis=-1, keepdims=True)
    return (x - mu) * lax.rsqrt(var + 1e-6) * s + b


def _dot(a, b):
    return jnp.dot(a, b, preferred_element_type=jnp.float32)


def _make_gather(N, D, E):
    EW = E // NS
    NCH = EW // G
    NG = NCH // KG
    GR = KG * G

    mesh = plsc.VectorSubcoreMesh(core_axis_name="c", subcore_axis_name="s")

    @functools.partial(
        pl.kernel,
        out_type=jax.ShapeDtypeStruct((2, E, D), jnp.float32),
        mesh=mesh,
        scratch_types=[
            pltpu.VMEM((NCH, G), jnp.int32),
            pltpu.VMEM((3, GR, D), jnp.float32),
            pltpu.SemaphoreType.DMA((3,)),
            pltpu.SemaphoreType.DMA((3,)),
        ],
    )
    def gather_k(p_hbm, idx_hbm, sr_out, idxv, big, sem_g, sem_o):
        cid = lax.axis_index("c")
        sid = lax.axis_index("s")
        base = sid * EW
        pltpu.sync_copy(idx_hbm.at[cid, sid], idxv)
        src = p_hbm.at[cid]

        def _fire_gathers(g, buf):
            for k in range(KG):
                pltpu.async_copy(
                    src.at[idxv.at[g * KG + k]],
                    big.at[buf, pl.ds(k * G, G)],
                    sem_g.at[buf],
                )

        def _wait_gathers(buf):
            for _ in range(KG):
                pltpu.make_async_copy(
                    src.at[idxv.at[0]],
                    big.at[buf, pl.ds(0, G)],
                    sem_g.at[buf],
                ).wait()

        def _fire_out(g, buf):
            pltpu.async_copy(
                big.at[buf],
                sr_out.at[cid, pl.ds(base + g * GR, GR)],
                sem_o.at[buf],
            )

        def _drain_out(buf):
            pltpu.make_async_copy(
                big.at[buf], sr_out.at[cid, pl.ds(base, GR)], sem_o.at[buf],
            ).wait()

        def body(g, carry):
            cur = lax.rem(g, 3)
            prv = lax.rem(g + 2, 3)

            @pl.when(g >= 3)
            def _():
                _drain_out(cur)

            _fire_gathers(g, cur)

            @pl.when(g >= 1)
            def _():
                _wait_gathers(prv)
                _fire_out(g - 1, prv)
            return carry

        lax.fori_loop(0, NG, body, 0)
        last = (NG - 1) % 3
        _wait_gathers(last)
        _fire_out(NG - 1, last)
        for b in ((NG - 3) % 3, (NG - 2) % 3, last):
            _drain_out(b)

    return gather_k


def _make_scatter(D, E, NPAD):
    EW = E // NW
    NCH = EW // G
    RPT = NPAD // NS
    mesh = plsc.VectorSubcoreMesh(core_axis_name="c", subcore_axis_name="s")

    @functools.partial(
        pl.kernel,
        out_type=(jax.ShapeDtypeStruct((NPAD, D), jnp.float32),
                  jax.ShapeDtypeStruct((NPAD, D), jnp.float32)),
        mesh=mesh,
        scratch_types=[
            pltpu.VMEM((3, G), jnp.int32),
            pltpu.VMEM((3, G, D), jnp.float32),
            pltpu.VMEM_SHARED((NPAD, D), jnp.float32),
            pltpu.SemaphoreType.DMA((3,)),
            pltpu.SemaphoreType.DMA((3,)),
        ],
    )
    def scatter_k(msg_hbm, ridx_hbm, zeros_hbm, agg0, agg1,
                  rivc, big, shared, sem_l, sem_sc):
        cid = lax.axis_index("c")
        sid = lax.axis_index("s")
        wid = sid * NC + cid
        base = wid * EW
        pltpu.sync_copy(zeros_hbm.at[pl.ds(sid * RPT, RPT)],
                        shared.at[pl.ds(sid * RPT, RPT)])
        plsc.subcore_barrier()

        def _load(g, buf):
            pltpu.async_copy(msg_hbm.at[pl.ds(base + g * G, G)],
                             big.at[buf], sem_l.at[buf])
            pltpu.async_copy(ridx_hbm.at[wid, g], rivc.at[buf],
                             sem_l.at[buf])

        def _wait_load(buf):
            pltpu.make_async_copy(
                msg_hbm.at[pl.ds(base, G)], big.at[buf],
                sem_l.at[buf]).wait()
            pltpu.make_async_copy(
                ridx_hbm.at[wid, 0], rivc.at[buf], sem_l.at[buf]).wait()

        def _drain(buf):
            pltpu.make_async_copy(
                big.at[buf], shared.at[rivc.at[buf]], sem_sc.at[buf]).wait()

        _load(0, 0)
        _load(1, 1)

        def body(g, carry):
            cur = lax.rem(g, 3)
            nxt = lax.rem(g + 1, 3)
            _wait_load(cur)
            pltpu.async_copy(big.at[cur], shared.at[rivc.at[cur]],
                             sem_sc.at[cur], add=True)

            @pl.when(g >= 2)
            def _():
                _drain(nxt)

            @pl.when(g + 2 < NCH)
            def _():
                _load(g + 2, nxt)
            return carry

        lax.fori_loop(0, NCH, body, 0)
        _drain((NCH - 2) % 3)
        _drain((NCH - 1) % 3)
        plsc.subcore_barrier()

        @pl.when(cid == 0)
        def _():
            pltpu.sync_copy(shared.at[pl.ds(sid * RPT, RPT)],
                            agg0.at[pl.ds(sid * RPT, RPT)])

        @pl.when(cid == 1)
        def _():
            pltpu.sync_copy(shared.at[pl.ds(sid * RPT, RPT)],
                            agg1.at[pl.ds(sid * RPT, RPT)])

    return scatter_k


def _make_precompute(N, D, H, BN):
    def body(n_ref, w0a, w0b, b0, out_ref):
        n = n_ref[...]
        out_ref[0] = _dot(n, w0a[...]) + b0[...]
        out_ref[1] = _dot(n, w0b[...])

    rep = lambda i: (0, 0)
    return pl.pallas_call(
        body,
        grid=(N // BN,),
        in_specs=[
            pl.BlockSpec((BN, D), lambda i: (i, 0)),
            pl.BlockSpec((D, H), rep),
            pl.BlockSpec((D, H), rep),
            pl.BlockSpec((1, H), rep),
        ],
        out_specs=pl.BlockSpec((2, BN, H), lambda i: (0, i, 0)),
        out_shape=jax.ShapeDtypeStruct((2, N, H), jnp.float32),
    )


def _make_edge_mlp(E, D, DE, H, BE):
    def body(s_ref, r_ref, et_ref, w0c, w1, b1, lns, lnb, out_ref):
        x = s_ref[0] + r_ref[0]
        x = x + lax.dot_general(et_ref[...], w0c[...],
                                (((0,), (0,)), ((), ())),
                                preferred_element_type=jnp.float32)
        x = jnp.maximum(x, 0.0)
        m = _dot(x, w1[...]) + b1[...]
        out_ref[...] = _layernorm(m, lns[...], lnb[...])

    rep = lambda i: (0, 0)
    return pl.pallas_call(
        body,
        grid=(E // BE,),
        in_specs=[
            pl.BlockSpec((1, BE, H), lambda i: (0, i, 0)),
            pl.BlockSpec((1, BE, H), lambda i: (1, i, 0)),
            pl.BlockSpec((DE, BE), lambda i: (0, i)),
            pl.BlockSpec((DE, H), rep),
            pl.BlockSpec((H, H), rep),
            pl.BlockSpec((1, H), rep),
            pl.BlockSpec((1, H), rep),
            pl.BlockSpec((1, H), rep),
        ],
        out_specs=pl.BlockSpec((BE, H), lambda i: (i, 0)),
        out_shape=jax.ShapeDtypeStruct((E, H), jnp.float32),
    )


def _make_node_mlp(N, D, H, BN, with_p):
    def body(n_ref, a0_ref, a1_ref, w0a, w0b, b0, w1, b1, lns, lnb, *rest):
        agg = a0_ref[...] + a1_ref[...]
        x = _dot(n_ref[...], w0a[...]) + _dot(agg, w0b[...])
        x = jnp.maximum(x + b0[...], 0.0)
        u = _dot(x, w1[...]) + b1[...]
        h = n_ref[...] + _layernorm(u, lns[...], lnb[...])
        if with_p:
            pw0a, pw0b, pb0, out_ref, p_ref = rest
            p_ref[0] = _dot(h, pw0a[...]) + pb0[...]
            p_ref[1] = _dot(h, pw0b[...])
        else:
            (out_ref,) = rest
        out_ref[...] = h

    rep = lambda i: (0, 0)
    in_specs = [
        pl.BlockSpec((BN, D), lambda i: (i, 0)),
        pl.BlockSpec((BN, D), lambda i: (i, 0)),
        pl.BlockSpec((BN, D), lambda i: (i, 0)),
        pl.BlockSpec((D, H), rep),
        pl.BlockSpec((H, H), rep),
        pl.BlockSpec((1, H), rep),
        pl.BlockSpec((H, D), rep),
        pl.BlockSpec((1, D), rep),
        pl.BlockSpec((1, D), rep),
        pl.BlockSpec((1, D), rep),
    ]
    out_specs = pl.BlockSpec((BN, D), lambda i: (i, 0))
    out_shape = jax.ShapeDtypeStruct((N, D), jnp.float32)
    if with_p:
        in_specs += [
            pl.BlockSpec((D, H), rep),
            pl.BlockSpec((D, H), rep),
            pl.BlockSpec((1, H), rep),
        ]
        out_specs = [out_specs, pl.BlockSpec((2, BN, H), lambda i: (0, i, 0))]
        out_shape = [out_shape, jax.ShapeDtypeStruct((2, N, H), jnp.float32)]
    return pl.pallas_call(
        body,
        grid=(N // BN,),
        in_specs=in_specs,
        out_specs=out_specs,
        out_shape=out_shape,
    )


def kernel(nodes, edges, senders, receivers,
           enc_msg_W0, enc_msg_b0, enc_msg_W1, enc_msg_b1, enc_ln_m_s, enc_ln_m_b,
           enc_node_W0, enc_node_b0, enc_node_W1, enc_node_b1, enc_ln_n_s, enc_ln_n_b,
           prc_msg_W0, prc_msg_b0, prc_msg_W1, prc_msg_b1, prc_ln_m_s, prc_ln_m_b,
           prc_node_W0, prc_node_b0, prc_node_W1, prc_node_b1, prc_ln_n_s, prc_ln_n_b):
    N, D = nodes.shape
    E, DE = edges.shape
    H = enc_msg_W0.shape[1]

    EW = E // NW
    NCH = EW // G
    RPT = -(-N // NS)
    RPT = ((RPT + 7) // 8) * 8
    NPAD = RPT * NS

    idx4 = jnp.stack([senders, receivers]).reshape(2, NS, (E // NS) // G, G)
    r3 = receivers.reshape(NW, NCH, G)
    zeros = jnp.zeros((NPAD, D), jnp.float32)
    edgesT = edges.T

    pre_k = _make_precompute(N, D, H, BN=2000)
    gather_k = _make_gather(N, D, E)
    scatter_k = _make_scatter(D, E, NPAD)
    edge_k = _make_edge_mlp(E, D, DE, H, BE=6400)
    node_k = _make_node_mlp(N, D, H, 2000, with_p=False)
    node_kp = _make_node_mlp(N, D, H, 2000, with_p=True)

    P1 = pre_k(nodes, enc_msg_W0[:D], enc_msg_W0[D:2 * D],
               enc_msg_b0.reshape(1, H))
    SR = gather_k(P1, idx4)
    M = edge_k(SR, SR, edgesT, enc_msg_W0[2 * D:], enc_msg_W1,
               enc_msg_b1.reshape(1, H), enc_ln_m_s.reshape(1, H),
               enc_ln_m_b.reshape(1, H))
    agg0, agg1 = scatter_k(M, r3, zeros)
    h, P2 = node_kp(nodes, agg0, agg1, enc_node_W0[:D], enc_node_W0[D:],
                    enc_node_b0.reshape(1, H), enc_node_W1,
                    enc_node_b1.reshape(1, D), enc_ln_n_s.reshape(1, D),
                    enc_ln_n_b.reshape(1, D),
                    prc_msg_W0[:D], prc_msg_W0[D:2 * D],
                    prc_msg_b0.reshape(1, H))

    SR2 = gather_k(P2, idx4)
    M2 = edge_k(SR2, SR2, edgesT, prc_msg_W0[2 * D:], prc_msg_W1,
                prc_msg_b1.reshape(1, H), prc_ln_m_s.reshape(1, H),
                prc_ln_m_b.reshape(1, H))
    agg0b, agg1b = scatter_k(M2, r3, zeros)
    return node_k(h, agg0b, agg1b, prc_node_W0[:D], prc_node_W0[D:],
                  prc_node_b0.reshape(1, H), prc_node_W1,
                  prc_node_b1.reshape(1, D), prc_ln_n_s.reshape(1, D),
                  prc_ln_n_b.reshape(1, D))
